# baseline (speedup 1.0000x reference)
import functools

import jax
import jax.numpy as jnp
from jax import lax
from jax.experimental import pallas as pl
from jax.experimental.pallas import tpu as pltpu
from jax.experimental.pallas import tpu_sc as plsc

O_NODES = 10000
T_EDGES = 320000
EMB = 64
GD = 128
GH = 128
NUM_LAYERS = 5

NC = 2
NS = 16
LANES = 16
K = 128
EDGES_PER_TILE = T_EDGES // NS
FULL_CHUNKS = EDGES_PER_TILE // K
TAIL = EDGES_PER_TILE - FULL_CHUNKS * K
ROWS_PER_TILE = 632
OPAD = NS * ROWS_PER_TILE


def _mesh():
    return plsc.VectorSubcoreMesh(
        core_axis_name="c", subcore_axis_name="s", num_cores=NC, num_subcores=NS)


def _edge_pass(table_s, table_o, gidx_s, gidx_o, scat_s, scat_o, h, zeros,
               has_h, want_r, distinct_scatter, tag):
    out_type = [jax.ShapeDtypeStruct((NC, OPAD, GH), jnp.float32)]
    if want_r:
        out_type.append(jax.ShapeDtypeStruct((T_EDGES, GH), jnp.float32))
    scratch = [
        pltpu.VMEM_SHARED((OPAD, GH), jnp.float32),
        pltpu.VMEM((K,), jnp.int32),
        pltpu.VMEM((K,), jnp.int32),
        pltpu.VMEM((TAIL,), jnp.int32),
        pltpu.VMEM((TAIL,), jnp.int32),
        pltpu.VMEM((K, GH), jnp.float32),
        pltpu.VMEM((K, GH), jnp.float32),
        pltpu.VMEM((K, GH), jnp.float32),
        pltpu.SemaphoreType.DMA,
    ]
    if distinct_scatter:
        scratch.append(pltpu.VMEM((K,), jnp.int32))
        scratch.append(pltpu.VMEM((TAIL,), jnp.int32))

    def body(*refs):
        it = iter(refs)
        ts_hbm = next(it)
        to_hbm = next(it)
        gs_hbm = next(it)
        go_hbm = next(it)
        if distinct_scatter:
            ss_hbm = next(it)
            so_hbm = next(it)
        if has_h:
            h_hbm = next(it)
        z_hbm = next(it)
        sq_out = next(it)
        r_out = next(it) if want_r else None
        acc = next(it)
        gsI = next(it)
        goI = next(it)
        gsIt = next(it)
        goIt = next(it)
        gs_v = next(it)
        go_v = next(it)
        r_v = next(it)
        sem = next(it)
        if distinct_scatter:
            scI = next(it)
            scIt = next(it)

        c = lax.axis_index("c")
        w = lax.axis_index("s")
        base_rows = w * ROWS_PER_TILE
        pltpu.sync_copy(z_hbm.at[pl.ds(base_rows, ROWS_PER_TILE)],
                        acc.at[pl.ds(base_rows, ROWS_PER_TILE)])
        plsc.subcore_barrier()

        def process(base, n, giS, giO, sciS, sciO):
            pltpu.sync_copy(gs_hbm.at[pl.ds(base, n)], giS)
            pltpu.sync_copy(go_hbm.at[pl.ds(base, n)], giO)
            if distinct_scatter:
                @pl.when(c == 0)
                def _():
                    pltpu.sync_copy(ss_hbm.at[pl.ds(base, n)], sciS)

                @pl.when(c == 1)
                def _():
                    pltpu.sync_copy(so_hbm.at[pl.ds(base, n)], sciS)
            cp1 = pltpu.async_copy(ts_hbm.at[giS], gs_v.at[pl.ds(0, n)], sem)
            cp2 = pltpu.async_copy(to_hbm.at[giO], go_v.at[pl.ds(0, n)], sem)
            if has_h:
                pltpu.sync_copy(h_hbm.at[pl.ds(base, n)], r_v.at[pl.ds(0, n)])
            cp1.wait()
            cp2.wait()

            def row(i, _):
                for j in range(GH // LANES):
                    sl = pl.ds(j * LANES, LANES)
                    a = gs_v[i, sl] + go_v[i, sl]
                    if has_h:
                        a = a + r_v[i, sl]
                    r_v[i, sl] = jnp.maximum(a, 0.0)
                return 0
            lax.fori_loop(0, n, row, 0)

            if want_r:
                @pl.when(c == 0)
                def _():
                    pltpu.sync_copy(r_v.at[pl.ds(0, n)], r_out.at[pl.ds(base, n)])
            if distinct_scatter:
                pltpu.sync_copy(r_v.at[pl.ds(0, n)], acc.at[sciS], add=True)
            else:
                @pl.when(c == 0)
                def _():
                    pltpu.sync_copy(r_v.at[pl.ds(0, n)], acc.at[giS], add=True)

                @pl.when(c == 1)
                def _():
                    pltpu.sync_copy(r_v.at[pl.ds(0, n)], acc.at[giO], add=True)

        def chunk(i, _):
            process(w * EDGES_PER_TILE + i * K, K, gsI, goI,
                    scI if distinct_scatter else None, None)
            return 0
        lax.fori_loop(0, FULL_CHUNKS, chunk, 0)
        process(w * EDGES_PER_TILE + FULL_CHUNKS * K, TAIL, gsIt, goIt,
                scIt if distinct_scatter else None, None)

        plsc.subcore_barrier()
        pltpu.sync_copy(acc.at[pl.ds(base_rows, ROWS_PER_TILE)],
                        sq_out.at[c, pl.ds(base_rows, ROWS_PER_TILE)])

    kfn = functools.partial(
        pl.kernel, mesh=_mesh(), out_type=tuple(out_type),
        scratch_types=scratch, name=f"sc_edge_pass_{tag}")(body)

    args = [table_s, table_o, gidx_s, gidx_o]
    if distinct_scatter:
        args += [scat_s, scat_o]
    if has_h:
        args.append(h)
    args.append(zeros)
    return kfn(*args)


def _h_matmul(r, M, cvec):
    BT = 2000

    def body(r_ref, m_ref, c_ref, o_ref):
        o_ref[...] = jnp.dot(r_ref[...], m_ref[...],
                             preferred_element_type=jnp.float32) + c_ref[...]

    return pl.pallas_call(
        body,
        grid=(T_EDGES // BT,),
        in_specs=[
            pl.BlockSpec((BT, GH), lambda i: (i, 0)),
            pl.BlockSpec((GH, GH), lambda i: (0, 0)),
            pl.BlockSpec((1, GH), lambda i: (0, 0)),
        ],
        out_specs=pl.BlockSpec((BT, GH), lambda i: (i, 0)),
        out_shape=jax.ShapeDtypeStruct((T_EDGES, GH), jnp.float32),
        name="tc_h_matmul",
    )(r, M, cvec.reshape(1, GH))


def _node_update(S, Q, csb, cob, W2a, W2c, b2a, b2c, V1, d1, V2, d2,
                 WA, bA, WB, bB, final, tag):
    BO = 2000

    def body(s_ref, q_ref, cs_ref, co_ref, w2a, w2c, vb2a, vb2c,
             v1, vd1, v2, vd2, wa, vba, wb, vbb, *outs):
        cs = cs_ref[...]
        co = co_ref[...]
        pooled = (jnp.dot(s_ref[...], w2a[...], preferred_element_type=jnp.float32)
                  + jnp.dot(q_ref[...], w2c[...], preferred_element_type=jnp.float32)
                  + cs * vb2a[...] + co * vb2c[...])
        pooled = pooled / jnp.maximum(cs + co, 1.0)
        t = jax.nn.relu(jnp.dot(pooled, v1[...], preferred_element_type=jnp.float32)
                        + vd1[...])
        obj = jnp.dot(t, v2[...], preferred_element_type=jnp.float32) + vd2[...]
        if final:
            t2 = jax.nn.relu(jnp.dot(obj, wa[...], preferred_element_type=jnp.float32)
                             + vba[...])
            outs[0][...] = jnp.dot(t2, wb[...], preferred_element_type=jnp.float32) + vbb[...]
        else:
            outs[0][...] = jnp.dot(obj, wa[...], preferred_element_type=jnp.float32)
            outs[1][...] = jnp.dot(obj, wb[...], preferred_element_type=jnp.float32)

    mat = lambda: pl.BlockSpec((GH, GH), lambda i: (0, 0))
    vec = lambda: pl.BlockSpec((1, GH), lambda i: (0, 0))
    big = lambda: pl.BlockSpec((BO, GH), lambda i: (i, 0))
    n_out = 1 if final else 2
    out = pl.pallas_call(
        body,
        grid=(O_NODES // BO,),
        in_specs=[big(), big(), big(), big(), mat(), mat(), vec(), vec(),
                  mat(), vec(), mat(), vec(), mat(), vec(), mat(), vec()],
        out_specs=tuple(big() for _ in range(n_out)),
        out_shape=tuple(jax.ShapeDtypeStruct((O_NODES, GH), jnp.float32)
                        for _ in range(n_out)),
        name=f"tc_node_update_{tag}",
    )(S, Q, csb, cob, W2a, W2c, b2a.reshape(1, GH), b2c.reshape(1, GH),
      V1, d1.reshape(1, GH), V2, d2.reshape(1, GH),
      WA, bA.reshape(1, GH), WB, bB.reshape(1, GH))
    return out


def kernel(objs, triples, params):
    s = triples[:, 0]
    p = triples[:, 1]
    o = triples[:, 2]
    gconv = params["gconv"]

    (Wa0, ba0), (Wb0, bb0) = gconv[0]["net1"]
    W1s0, W1p0, W1o0 = Wa0[:EMB], Wa0[EMB:2 * EMB], Wa0[2 * EMB:]
    As = params["obj_emb"] @ W1s0
    Pp = params["pred_emb"] @ W1p0
    C = (As[:, None, :] + Pp[None, :, :]).reshape(-1, GH) + ba0
    Ao = params["obj_emb"] @ W1o0

    Wsplit = []
    for ell in range(NUM_LAYERS):
        (Wa, ba), (Wb, bb) = gconv[ell]["net1"]
        din = EMB if ell == 0 else GD
        Wsplit.append(dict(
            W1s=Wa[:din], W1p=Wa[din:2 * din], W1o=Wa[2 * din:], b1=ba,
            W2a=Wb[:, :GH], W2b=Wb[:, GH:GH + GD], W2c=Wb[:, GH + GD:],
            b2a=bb[:GH], b2b=bb[GH:GH + GD], b2c=bb[GH + GD:]))

    gidx0_s = jnp.take(objs, s) * 16 + p
    gidx0_o = jnp.take(objs, o)

    zerosGH = jnp.zeros((OPAD, GH), jnp.float32)

    half = jnp.full((8, GH), 0.5, jnp.float32)
    zidx = jnp.zeros((T_EDGES,), jnp.int32)
    (CNT,) = _edge_pass(half, half, zidx, zidx, s, o, None, zerosGH,
                        has_h=False, want_r=False, distinct_scatter=True,
                        tag="cnt")
    csb = CNT[0][:O_NODES]
    cob = CNT[1][:O_NODES]

    SQ, r = _edge_pass(C, Ao, gidx0_s, gidx0_o, s, o, None, zerosGH,
                       has_h=False, want_r=True, distinct_scatter=True, tag="l0")

    for ell in range(1, NUM_LAYERS + 1):
        prev = Wsplit[ell - 1]
        if ell < NUM_LAYERS:
            cur = Wsplit[ell]
            Bs, Bo = _node_update(
                SQ[0][:O_NODES], SQ[1][:O_NODES], csb, cob, prev["W2a"], prev["W2c"],
                prev["b2a"], prev["b2c"],
                gconv[ell - 1]["net2"][0][0], gconv[ell - 1]["net2"][0][1],
                gconv[ell - 1]["net2"][1][0], gconv[ell - 1]["net2"][1][1],
                cur["W1s"], jnp.zeros((GH,), jnp.float32),
                cur["W1o"], jnp.zeros((GH,), jnp.float32),
                final=False, tag=f"l{ell - 1}")
            M = prev["W2b"] @ cur["W1p"]
            cvec = cur["b1"] + prev["b2b"] @ cur["W1p"]
            h = _h_matmul(r, M, cvec)
            want_r = ell < NUM_LAYERS - 1
            res = _edge_pass(Bs, Bo, s, o, None, None, h, zerosGH,
                             has_h=True, want_r=want_r,
                             distinct_scatter=False, tag=f"l{ell}")
            if want_r:
                SQ, r = res
            else:
                (SQ,) = res
        else:
            (U1, e1), (U2, e2) = params["box"]
            U2p = jnp.zeros((GH, GH), jnp.float32).at[:, :4].set(U2)
            e2p = jnp.zeros((GH,), jnp.float32).at[:4].set(e2)
            (boxes_pad,) = _node_update(
                SQ[0][:O_NODES], SQ[1][:O_NODES], csb, cob, prev["W2a"], prev["W2c"],
                prev["b2a"], prev["b2c"],
                gconv[ell - 1]["net2"][0][0], gconv[ell - 1]["net2"][0][1],
                gconv[ell - 1]["net2"][1][0], gconv[ell - 1]["net2"][1][1],
                U1, e1, U2p, e2p, final=True, tag="box")
            return boxes_pad[:, :4]


# speedup vs baseline: 5.7790x; 5.7790x over previous
import functools

import jax
import jax.numpy as jnp
from jax import lax
from jax.experimental import pallas as pl
from jax.experimental.pallas import tpu as pltpu
from jax.experimental.pallas import tpu_sc as plsc

O_NODES = 10000
T_EDGES = 320000
EMB = 64
GD = 128
GH = 128
NUM_LAYERS = 5

NC = 2
NS = 16
LANES = 16
K = 128
EDGES_PER_TILE = T_EDGES // NS
FULL_CHUNKS = EDGES_PER_TILE // K
TAIL = EDGES_PER_TILE - FULL_CHUNKS * K
ROWS_PER_TILE = 632
OPAD = NS * ROWS_PER_TILE


def _mesh():
    return plsc.VectorSubcoreMesh(
        core_axis_name="c", subcore_axis_name="s", num_cores=NC, num_subcores=NS)


def _edge_pass(table_s, table_o, gidx_s, gidx_o, scat_s, scat_o, h, zeros,
               has_h, want_r, distinct_scatter, tag, want_counts=False,
               zeros16=None, ones16=None):
    out_type = [jax.ShapeDtypeStruct((NC, OPAD, GH), jnp.float32)]
    if want_r:
        out_type.append(jax.ShapeDtypeStruct((T_EDGES, GH), jnp.float32))
    if want_counts:
        out_type.append(jax.ShapeDtypeStruct((NC, OPAD, LANES), jnp.float32))
    scratch = [
        pltpu.VMEM_SHARED((OPAD, GH), jnp.float32),
        pltpu.VMEM((K,), jnp.int32),
        pltpu.VMEM((K,), jnp.int32),
        pltpu.VMEM((TAIL,), jnp.int32),
        pltpu.VMEM((TAIL,), jnp.int32),
        pltpu.VMEM((K, GH), jnp.float32),
        pltpu.VMEM((K, GH), jnp.float32),
        pltpu.VMEM((K, GH), jnp.float32),
        pltpu.SemaphoreType.DMA,
    ]
    if distinct_scatter:
        scratch.append(pltpu.VMEM((K,), jnp.int32))
        scratch.append(pltpu.VMEM((TAIL,), jnp.int32))
    if want_counts:
        scratch.append(pltpu.VMEM_SHARED((OPAD, LANES), jnp.float32))
        scratch.append(pltpu.VMEM((K, LANES), jnp.float32))

    def body(*refs):
        it = iter(refs)
        ts_hbm = next(it)
        to_hbm = next(it)
        gs_hbm = next(it)
        go_hbm = next(it)
        if distinct_scatter:
            ss_hbm = next(it)
            so_hbm = next(it)
        if has_h:
            h_hbm = next(it)
        z_hbm = next(it)
        if want_counts:
            z16_hbm = next(it)
            ones_hbm = next(it)
        sq_out = next(it)
        r_out = next(it) if want_r else None
        cnt_out = next(it) if want_counts else None
        acc = next(it)
        gsI = next(it)
        goI = next(it)
        gsIt = next(it)
        goIt = next(it)
        gs_v = next(it)
        go_v = next(it)
        r_v = next(it)
        sem = next(it)
        if distinct_scatter:
            scI = next(it)
            scIt = next(it)
        if want_counts:
            acc2 = next(it)
            ones_v = next(it)

        c = lax.axis_index("c")
        w = lax.axis_index("s")
        base_rows = w * ROWS_PER_TILE
        pltpu.sync_copy(z_hbm.at[pl.ds(base_rows, ROWS_PER_TILE)],
                        acc.at[pl.ds(base_rows, ROWS_PER_TILE)])
        if want_counts:
            pltpu.sync_copy(z16_hbm.at[pl.ds(base_rows, ROWS_PER_TILE)],
                            acc2.at[pl.ds(base_rows, ROWS_PER_TILE)])
            pltpu.sync_copy(ones_hbm, ones_v)
        plsc.subcore_barrier()

        def process(base, n, giS, giO, sciS, sciO):
            pltpu.sync_copy(gs_hbm.at[pl.ds(base, n)], giS)
            pltpu.sync_copy(go_hbm.at[pl.ds(base, n)], giO)
            if distinct_scatter:
                @pl.when(c == 0)
                def _():
                    pltpu.sync_copy(ss_hbm.at[pl.ds(base, n)], sciS)

                @pl.when(c == 1)
                def _():
                    pltpu.sync_copy(so_hbm.at[pl.ds(base, n)], sciS)
            cp1 = pltpu.async_copy(ts_hbm.at[giS], gs_v.at[pl.ds(0, n)], sem)
            cp2 = pltpu.async_copy(to_hbm.at[giO], go_v.at[pl.ds(0, n)], sem)
            if has_h:
                pltpu.sync_copy(h_hbm.at[pl.ds(base, n)], r_v.at[pl.ds(0, n)])
            cp1.wait()
            cp2.wait()

            def row(i, _):
                for j in range(GH // LANES):
                    sl = pl.ds(j * LANES, LANES)
                    a = gs_v[i, sl] + go_v[i, sl]
                    if has_h:
                        a = a + r_v[i, sl]
                    r_v[i, sl] = jnp.maximum(a, 0.0)
                return 0
            lax.fori_loop(0, n, row, 0)

            if want_r:
                @pl.when(c == 0)
                def _():
                    pltpu.sync_copy(r_v.at[pl.ds(0, n)], r_out.at[pl.ds(base, n)])
            if distinct_scatter:
                pltpu.sync_copy(r_v.at[pl.ds(0, n)], acc.at[sciS], add=True)
                if want_counts:
                    pltpu.sync_copy(ones_v.at[pl.ds(0, n)], acc2.at[sciS],
                                    add=True)
            else:
                @pl.when(c == 0)
                def _():
                    pltpu.sync_copy(r_v.at[pl.ds(0, n)], acc.at[giS], add=True)

                @pl.when(c == 1)
                def _():
                    pltpu.sync_copy(r_v.at[pl.ds(0, n)], acc.at[giO], add=True)

        def chunk(i, _):
            process(w * EDGES_PER_TILE + i * K, K, gsI, goI,
                    scI if distinct_scatter else None, None)
            return 0
        lax.fori_loop(0, FULL_CHUNKS, chunk, 0)
        process(w * EDGES_PER_TILE + FULL_CHUNKS * K, TAIL, gsIt, goIt,
                scIt if distinct_scatter else None, None)

        plsc.subcore_barrier()
        pltpu.sync_copy(acc.at[pl.ds(base_rows, ROWS_PER_TILE)],
                        sq_out.at[c, pl.ds(base_rows, ROWS_PER_TILE)])
        if want_counts:
            pltpu.sync_copy(acc2.at[pl.ds(base_rows, ROWS_PER_TILE)],
                            cnt_out.at[c, pl.ds(base_rows, ROWS_PER_TILE)])

    kfn = functools.partial(
        pl.kernel, mesh=_mesh(), out_type=tuple(out_type),
        scratch_types=scratch, name=f"sc_edge_pass_{tag}")(body)

    args = [table_s, table_o, gidx_s, gidx_o]
    if distinct_scatter:
        args += [scat_s, scat_o]
    if has_h:
        args.append(h)
    args.append(zeros)
    if want_counts:
        args += [zeros16, ones16]
    return kfn(*args)


def _count_pass(scat_s, scat_o, zerosGH, onesGH):
    out_type = (jax.ShapeDtypeStruct((NC, OPAD, GH), jnp.float32),)
    scratch = [
        pltpu.VMEM_SHARED((OPAD, GH), jnp.float32),
        pltpu.VMEM((K,), jnp.int32),
        pltpu.VMEM((TAIL,), jnp.int32),
        pltpu.VMEM((K, GH), jnp.float32),
    ]

    def body(ss_hbm, so_hbm, z_hbm, ones_hbm, cnt_out, acc, scI, scIt, ones_v):
        c = lax.axis_index("c")
        w = lax.axis_index("s")
        base_rows = w * ROWS_PER_TILE
        pltpu.sync_copy(z_hbm.at[pl.ds(base_rows, ROWS_PER_TILE)],
                        acc.at[pl.ds(base_rows, ROWS_PER_TILE)])
        pltpu.sync_copy(ones_hbm, ones_v)
        plsc.subcore_barrier()

        def process(base, n, sci):
            @pl.when(c == 0)
            def _():
                pltpu.sync_copy(ss_hbm.at[pl.ds(base, n)], sci)

            @pl.when(c == 1)
            def _():
                pltpu.sync_copy(so_hbm.at[pl.ds(base, n)], sci)
            pltpu.sync_copy(ones_v.at[pl.ds(0, n)], acc.at[sci], add=True)

        def chunk(i, _):
            process(w * EDGES_PER_TILE + i * K, K, scI)
            return 0
        lax.fori_loop(0, FULL_CHUNKS, chunk, 0)
        process(w * EDGES_PER_TILE + FULL_CHUNKS * K, TAIL, scIt)

        plsc.subcore_barrier()
        pltpu.sync_copy(acc.at[pl.ds(base_rows, ROWS_PER_TILE)],
                        cnt_out.at[c, pl.ds(base_rows, ROWS_PER_TILE)])

    kfn = functools.partial(
        pl.kernel, mesh=_mesh(), out_type=out_type,
        scratch_types=scratch, name="sc_count_pass")(body)
    return kfn(scat_s, scat_o, zerosGH, onesGH)


def _h_matmul(r, M, cvec):
    BT = 2000

    def body(r_ref, m_ref, c_ref, o_ref):
        o_ref[...] = jnp.dot(r_ref[...], m_ref[...],
                             preferred_element_type=jnp.float32) + c_ref[...]

    return pl.pallas_call(
        body,
        grid=(T_EDGES // BT,),
        in_specs=[
            pl.BlockSpec((BT, GH), lambda i: (i, 0)),
            pl.BlockSpec((GH, GH), lambda i: (0, 0)),
            pl.BlockSpec((1, GH), lambda i: (0, 0)),
        ],
        out_specs=pl.BlockSpec((BT, GH), lambda i: (i, 0)),
        out_shape=jax.ShapeDtypeStruct((T_EDGES, GH), jnp.float32),
        name="tc_h_matmul",
    )(r, M, cvec.reshape(1, GH))


def _node_update(S, Q, csb, cob, W2a, W2c, b2a, b2c, V1, d1, V2, d2,
                 WA, bA, WB, bB, final, tag):
    BO = 2000

    def body(s_ref, q_ref, cs_ref, co_ref, w2a, w2c, vb2a, vb2c,
             v1, vd1, v2, vd2, wa, vba, wb, vbb, *outs):
        cs = cs_ref[...]
        co = co_ref[...]
        pooled = (jnp.dot(s_ref[...], w2a[...], preferred_element_type=jnp.float32)
                  + jnp.dot(q_ref[...], w2c[...], preferred_element_type=jnp.float32)
                  + cs * vb2a[...] + co * vb2c[...])
        pooled = pooled / jnp.maximum(cs + co, 1.0)
        t = jax.nn.relu(jnp.dot(pooled, v1[...], preferred_element_type=jnp.float32)
                        + vd1[...])
        obj = jnp.dot(t, v2[...], preferred_element_type=jnp.float32) + vd2[...]
        if final:
            t2 = jax.nn.relu(jnp.dot(obj, wa[...], preferred_element_type=jnp.float32)
                             + vba[...])
            outs[0][...] = jnp.dot(t2, wb[...], preferred_element_type=jnp.float32) + vbb[...]
        else:
            outs[0][...] = jnp.dot(obj, wa[...], preferred_element_type=jnp.float32)
            outs[1][...] = jnp.dot(obj, wb[...], preferred_element_type=jnp.float32)

    mat = lambda: pl.BlockSpec((GH, GH), lambda i: (0, 0))
    vec = lambda: pl.BlockSpec((1, GH), lambda i: (0, 0))
    big = lambda: pl.BlockSpec((BO, GH), lambda i: (i, 0))
    n_out = 1 if final else 2
    out = pl.pallas_call(
        body,
        grid=(O_NODES // BO,),
        in_specs=[big(), big(), big(), big(), mat(), mat(), vec(), vec(),
                  mat(), vec(), mat(), vec(), mat(), vec(), mat(), vec()],
        out_specs=tuple(big() for _ in range(n_out)),
        out_shape=tuple(jax.ShapeDtypeStruct((O_NODES, GH), jnp.float32)
                        for _ in range(n_out)),
        name=f"tc_node_update_{tag}",
    )(S, Q, csb, cob, W2a, W2c, b2a.reshape(1, GH), b2c.reshape(1, GH),
      V1, d1.reshape(1, GH), V2, d2.reshape(1, GH),
      WA, bA.reshape(1, GH), WB, bB.reshape(1, GH))
    return out


def kernel(objs, triples, params):
    s = triples[:, 0]
    p = triples[:, 1]
    o = triples[:, 2]
    gconv = params["gconv"]

    (Wa0, ba0), (Wb0, bb0) = gconv[0]["net1"]
    W1s0, W1p0, W1o0 = Wa0[:EMB], Wa0[EMB:2 * EMB], Wa0[2 * EMB:]
    As = params["obj_emb"] @ W1s0
    Pp = params["pred_emb"] @ W1p0
    C = (As[:, None, :] + Pp[None, :, :]).reshape(-1, GH) + ba0
    Ao = params["obj_emb"] @ W1o0

    Wsplit = []
    for ell in range(NUM_LAYERS):
        (Wa, ba), (Wb, bb) = gconv[ell]["net1"]
        din = EMB if ell == 0 else GD
        Wsplit.append(dict(
            W1s=Wa[:din], W1p=Wa[din:2 * din], W1o=Wa[2 * din:], b1=ba,
            W2a=Wb[:, :GH], W2b=Wb[:, GH:GH + GD], W2c=Wb[:, GH + GD:],
            b2a=bb[:GH], b2b=bb[GH:GH + GD], b2c=bb[GH + GD:]))

    gidx0_s = jnp.take(objs, s) * 16 + p
    gidx0_o = jnp.take(objs, o)

    zerosGH = jnp.zeros((OPAD, GH), jnp.float32)
    onesGH = jnp.ones((K, GH), jnp.float32)

    (CNT,) = _count_pass(s, o, zerosGH, onesGH)
    csb = CNT[0][:O_NODES]
    cob = CNT[1][:O_NODES]

    SQ, r = _edge_pass(C, Ao, gidx0_s, gidx0_o, s, o, None, zerosGH,
                       has_h=False, want_r=True, distinct_scatter=True,
                       tag="l0")

    for ell in range(1, NUM_LAYERS + 1):
        prev = Wsplit[ell - 1]
        if ell < NUM_LAYERS:
            cur = Wsplit[ell]
            Bs, Bo = _node_update(
                SQ[0][:O_NODES], SQ[1][:O_NODES], csb, cob, prev["W2a"], prev["W2c"],
                prev["b2a"], prev["b2c"],
                gconv[ell - 1]["net2"][0][0], gconv[ell - 1]["net2"][0][1],
                gconv[ell - 1]["net2"][1][0], gconv[ell - 1]["net2"][1][1],
                cur["W1s"], jnp.zeros((GH,), jnp.float32),
                cur["W1o"], jnp.zeros((GH,), jnp.float32),
                final=False, tag=f"l{ell - 1}")
            M = prev["W2b"] @ cur["W1p"]
            cvec = cur["b1"] + prev["b2b"] @ cur["W1p"]
            h = _h_matmul(r, M, cvec)
            want_r = ell < NUM_LAYERS - 1
            res = _edge_pass(Bs, Bo, s, o, None, None, h, zerosGH,
                             has_h=True, want_r=want_r,
                             distinct_scatter=False, tag=f"l{ell}")
            if want_r:
                SQ, r = res
            else:
                (SQ,) = res
        else:
            (U1, e1), (U2, e2) = params["box"]
            U2p = jnp.zeros((GH, GH), jnp.float32).at[:, :4].set(U2)
            e2p = jnp.zeros((GH,), jnp.float32).at[:4].set(e2)
            (boxes_pad,) = _node_update(
                SQ[0][:O_NODES], SQ[1][:O_NODES], csb, cob, prev["W2a"], prev["W2c"],
                prev["b2a"], prev["b2c"],
                gconv[ell - 1]["net2"][0][0], gconv[ell - 1]["net2"][0][1],
                gconv[ell - 1]["net2"][1][0], gconv[ell - 1]["net2"][1][1],
                U1, e1, U2p, e2p, final=True, tag="box")
            return boxes_pad[:, :4]


# speedup vs baseline: 9.2893x; 1.6074x over previous
import functools

import jax
import jax.numpy as jnp
from jax import lax
from jax.experimental import pallas as pl
from jax.experimental.pallas import tpu as pltpu
from jax.experimental.pallas import tpu_sc as plsc

O_NODES = 10000
T_EDGES = 320000
EMB = 64
GD = 128
GH = 128
NUM_LAYERS = 5

NC = 2
NS = 16
LANES = 16
K = 128
EDGES_PER_TILE = T_EDGES // NS
FULL_CHUNKS = EDGES_PER_TILE // K
HALF_CHUNKS = FULL_CHUNKS // 2
TAIL = EDGES_PER_TILE - FULL_CHUNKS * K
ROWS_PER_TILE = 632
OPAD = NS * ROWS_PER_TILE


def _mesh():
    return plsc.VectorSubcoreMesh(
        core_axis_name="c", subcore_axis_name="s", num_cores=NC, num_subcores=NS)


def _edge_pass(table_s, table_o, s_idx, o_idx, h, zeros, want_r, tag):
    out_type = [jax.ShapeDtypeStruct((NC, OPAD, GH), jnp.float32)]
    if want_r:
        out_type.append(jax.ShapeDtypeStruct((T_EDGES, GH), jnp.float32))
    scratch = [
        pltpu.VMEM_SHARED((OPAD, GH), jnp.float32),
        pltpu.VMEM((K,), jnp.int32),
        pltpu.VMEM((K,), jnp.int32),
        pltpu.VMEM((TAIL,), jnp.int32),
        pltpu.VMEM((TAIL,), jnp.int32),
        pltpu.VMEM((K, GH), jnp.float32),
    ]

    def body(*refs):
        it = iter(refs)
        ts_hbm = next(it)
        to_hbm = next(it)
        gs_hbm = next(it)
        go_hbm = next(it)
        h_hbm = next(it)
        z_hbm = next(it)
        sq_out = next(it)
        r_out = next(it) if want_r else None
        acc = next(it)
        gsI = next(it)
        goI = next(it)
        gsIt = next(it)
        goIt = next(it)
        r_v = next(it)

        c = lax.axis_index("c")
        w = lax.axis_index("s")
        base_rows = w * ROWS_PER_TILE
        pltpu.sync_copy(z_hbm.at[pl.ds(base_rows, ROWS_PER_TILE)],
                        acc.at[pl.ds(base_rows, ROWS_PER_TILE)])
        plsc.subcore_barrier()

        def process(base, n, giS, giO, write_r):
            pltpu.sync_copy(gs_hbm.at[pl.ds(base, n)], giS)
            pltpu.sync_copy(go_hbm.at[pl.ds(base, n)], giO)
            pltpu.sync_copy(h_hbm.at[pl.ds(base, n)], r_v.at[pl.ds(0, n)])
            pltpu.sync_copy(ts_hbm.at[giS], r_v.at[pl.ds(0, n)], add=True)
            pltpu.sync_copy(to_hbm.at[giO], r_v.at[pl.ds(0, n)], add=True)

            def row(i, _):
                for j in range(GH // LANES):
                    sl = pl.ds(j * LANES, LANES)
                    r_v[i, sl] = jnp.maximum(r_v[i, sl], 0.0)
                return 0
            lax.fori_loop(0, n, row, 0)

            if want_r:
                @pl.when(write_r)
                def _():
                    pltpu.sync_copy(r_v.at[pl.ds(0, n)], r_out.at[pl.ds(base, n)])

            @pl.when(c == 0)
            def _():
                pltpu.sync_copy(r_v.at[pl.ds(0, n)], acc.at[giS], add=True)

            @pl.when(c == 1)
            def _():
                pltpu.sync_copy(r_v.at[pl.ds(0, n)], acc.at[giO], add=True)

        def chunk(i, _):
            wr = jnp.where(c == 0, i < HALF_CHUNKS, i >= HALF_CHUNKS)
            process(w * EDGES_PER_TILE + i * K, K, gsI, goI, wr)
            return 0
        lax.fori_loop(0, FULL_CHUNKS, chunk, 0)
        process(w * EDGES_PER_TILE + FULL_CHUNKS * K, TAIL, gsIt, goIt, c == 1)

        plsc.subcore_barrier()
        pltpu.sync_copy(acc.at[pl.ds(base_rows, ROWS_PER_TILE)],
                        sq_out.at[c, pl.ds(base_rows, ROWS_PER_TILE)])

    kfn = functools.partial(
        pl.kernel, mesh=_mesh(), out_type=tuple(out_type),
        scratch_types=scratch, name=f"sc_edge_pass_{tag}")(body)
    return kfn(table_s, table_o, s_idx, o_idx, h, zeros)


def _count_pass(scat_s, scat_o, zerosGH, onesGH):
    out_type = (jax.ShapeDtypeStruct((NC, OPAD, GH), jnp.float32),)
    scratch = [
        pltpu.VMEM_SHARED((OPAD, GH), jnp.float32),
        pltpu.VMEM((K,), jnp.int32),
        pltpu.VMEM((TAIL,), jnp.int32),
        pltpu.VMEM((K, GH), jnp.float32),
    ]

    def body(ss_hbm, so_hbm, z_hbm, ones_hbm, cnt_out, acc, scI, scIt, ones_v):
        c = lax.axis_index("c")
        w = lax.axis_index("s")
        base_rows = w * ROWS_PER_TILE
        pltpu.sync_copy(z_hbm.at[pl.ds(base_rows, ROWS_PER_TILE)],
                        acc.at[pl.ds(base_rows, ROWS_PER_TILE)])
        pltpu.sync_copy(ones_hbm, ones_v)
        plsc.subcore_barrier()

        def process(base, n, sci):
            @pl.when(c == 0)
            def _():
                pltpu.sync_copy(ss_hbm.at[pl.ds(base, n)], sci)

            @pl.when(c == 1)
            def _():
                pltpu.sync_copy(so_hbm.at[pl.ds(base, n)], sci)
            pltpu.sync_copy(ones_v.at[pl.ds(0, n)], acc.at[sci], add=True)

        def chunk(i, _):
            process(w * EDGES_PER_TILE + i * K, K, scI)
            return 0
        lax.fori_loop(0, FULL_CHUNKS, chunk, 0)
        process(w * EDGES_PER_TILE + FULL_CHUNKS * K, TAIL, scIt)

        plsc.subcore_barrier()
        pltpu.sync_copy(acc.at[pl.ds(base_rows, ROWS_PER_TILE)],
                        cnt_out.at[c, pl.ds(base_rows, ROWS_PER_TILE)])

    kfn = functools.partial(
        pl.kernel, mesh=_mesh(), out_type=out_type,
        scratch_types=scratch, name="sc_count_pass")(body)
    return kfn(scat_s, scat_o, zerosGH, onesGH)


def _class_tables(objs_col, TsPad, ToPad):
    BO = 2000

    def body(c_ref, ts_ref, to_ref, as_ref, ao_ref):
        oh = (c_ref[...] == lax.broadcasted_iota(jnp.int32, (BO, GH), 1)
              ).astype(jnp.float32)
        as_ref[...] = jnp.dot(oh, ts_ref[...], preferred_element_type=jnp.float32)
        ao_ref[...] = jnp.dot(oh, to_ref[...], preferred_element_type=jnp.float32)

    return pl.pallas_call(
        body,
        grid=(O_NODES // BO,),
        in_specs=[
            pl.BlockSpec((BO, 1), lambda i: (i, 0)),
            pl.BlockSpec((GH, GH), lambda i: (0, 0)),
            pl.BlockSpec((GH, GH), lambda i: (0, 0)),
        ],
        out_specs=(pl.BlockSpec((BO, GH), lambda i: (i, 0)),
                   pl.BlockSpec((BO, GH), lambda i: (i, 0))),
        out_shape=(jax.ShapeDtypeStruct((O_NODES, GH), jnp.float32),
                   jax.ShapeDtypeStruct((O_NODES, GH), jnp.float32)),
        name="tc_class_tables",
    )(objs_col, TsPad, ToPad)


def _pred_h0(p_col, PpPad, bias):
    BT = 2000

    def body(p_ref, pp_ref, b_ref, o_ref):
        oh = (p_ref[...] == lax.broadcasted_iota(jnp.int32, (BT, GH), 1)
              ).astype(jnp.float32)
        o_ref[...] = jnp.dot(oh, pp_ref[...],
                             preferred_element_type=jnp.float32) + b_ref[...]

    return pl.pallas_call(
        body,
        grid=(T_EDGES // BT,),
        in_specs=[
            pl.BlockSpec((BT, 1), lambda i: (i, 0)),
            pl.BlockSpec((GH, GH), lambda i: (0, 0)),
            pl.BlockSpec((1, GH), lambda i: (0, 0)),
        ],
        out_specs=pl.BlockSpec((BT, GH), lambda i: (i, 0)),
        out_shape=jax.ShapeDtypeStruct((T_EDGES, GH), jnp.float32),
        name="tc_pred_h0",
    )(p_col, PpPad, bias.reshape(1, GH))


def _h_matmul(r, M, cvec):
    BT = 2000

    def body(r_ref, m_ref, c_ref, o_ref):
        o_ref[...] = jnp.dot(r_ref[...], m_ref[...],
                             preferred_element_type=jnp.float32) + c_ref[...]

    return pl.pallas_call(
        body,
        grid=(T_EDGES // BT,),
        in_specs=[
            pl.BlockSpec((BT, GH), lambda i: (i, 0)),
            pl.BlockSpec((GH, GH), lambda i: (0, 0)),
            pl.BlockSpec((1, GH), lambda i: (0, 0)),
        ],
        out_specs=pl.BlockSpec((BT, GH), lambda i: (i, 0)),
        out_shape=jax.ShapeDtypeStruct((T_EDGES, GH), jnp.float32),
        name="tc_h_matmul",
    )(r, M, cvec.reshape(1, GH))


def _node_update(S, Q, csb, cob, W2a, W2c, b2a, b2c, V1, d1, V2, d2,
                 WA, bA, WB, bB, final, tag):
    BO = 2000

    def body(s_ref, q_ref, cs_ref, co_ref, w2a, w2c, vb2a, vb2c,
             v1, vd1, v2, vd2, wa, vba, wb, vbb, *outs):
        cs = cs_ref[...]
        co = co_ref[...]
        pooled = (jnp.dot(s_ref[...], w2a[...], preferred_element_type=jnp.float32)
                  + jnp.dot(q_ref[...], w2c[...], preferred_element_type=jnp.float32)
                  + cs * vb2a[...] + co * vb2c[...])
        pooled = pooled / jnp.maximum(cs + co, 1.0)
        t = jax.nn.relu(jnp.dot(pooled, v1[...], preferred_element_type=jnp.float32)
                        + vd1[...])
        obj = jnp.dot(t, v2[...], preferred_element_type=jnp.float32) + vd2[...]
        if final:
            t2 = jax.nn.relu(jnp.dot(obj, wa[...], preferred_element_type=jnp.float32)
                             + vba[...])
            outs[0][...] = jnp.dot(t2, wb[...], preferred_element_type=jnp.float32) + vbb[...]
        else:
            outs[0][...] = jnp.dot(obj, wa[...], preferred_element_type=jnp.float32)
            outs[1][...] = jnp.dot(obj, wb[...], preferred_element_type=jnp.float32)

    mat = lambda: pl.BlockSpec((GH, GH), lambda i: (0, 0))
    vec = lambda: pl.BlockSpec((1, GH), lambda i: (0, 0))
    big = lambda: pl.BlockSpec((BO, GH), lambda i: (i, 0))
    n_out = 1 if final else 2
    out = pl.pallas_call(
        body,
        grid=(O_NODES // BO,),
        in_specs=[big(), big(), big(), big(), mat(), mat(), vec(), vec(),
                  mat(), vec(), mat(), vec(), mat(), vec(), mat(), vec()],
        out_specs=tuple(big() for _ in range(n_out)),
        out_shape=tuple(jax.ShapeDtypeStruct((O_NODES, GH), jnp.float32)
                        for _ in range(n_out)),
        name=f"tc_node_update_{tag}",
    )(S, Q, csb, cob, W2a, W2c, b2a.reshape(1, GH), b2c.reshape(1, GH),
      V1, d1.reshape(1, GH), V2, d2.reshape(1, GH),
      WA, bA.reshape(1, GH), WB, bB.reshape(1, GH))
    return out


def kernel(objs, triples, params):
    s = triples[:, 0]
    p = triples[:, 1]
    o = triples[:, 2]
    gconv = params["gconv"]

    (Wa0, ba0), (Wb0, bb0) = gconv[0]["net1"]
    W1s0, W1p0, W1o0 = Wa0[:EMB], Wa0[EMB:2 * EMB], Wa0[2 * EMB:]
    nclass = params["obj_emb"].shape[0]
    npred = params["pred_emb"].shape[0]
    TsPad = jnp.zeros((GH, GH), jnp.float32).at[:nclass].set(
        params["obj_emb"] @ W1s0)
    ToPad = jnp.zeros((GH, GH), jnp.float32).at[:nclass].set(
        params["obj_emb"] @ W1o0)
    PpPad = jnp.zeros((GH, GH), jnp.float32).at[:npred].set(
        params["pred_emb"] @ W1p0)

    Wsplit = []
    for ell in range(NUM_LAYERS):
        (Wa, ba), (Wb, bb) = gconv[ell]["net1"]
        din = EMB if ell == 0 else GD
        Wsplit.append(dict(
            W1s=Wa[:din], W1p=Wa[din:2 * din], W1o=Wa[2 * din:], b1=ba,
            W2a=Wb[:, :GH], W2b=Wb[:, GH:GH + GD], W2c=Wb[:, GH + GD:],
            b2a=bb[:GH], b2b=bb[GH:GH + GD], b2c=bb[GH + GD:]))

    zerosGH = jnp.zeros((OPAD, GH), jnp.float32)
    onesGH = jnp.ones((K, GH), jnp.float32)

    (CNT,) = _count_pass(s, o, zerosGH, onesGH)
    csb = CNT[0][:O_NODES]
    cob = CNT[1][:O_NODES]

    As_node, Ao_node = _class_tables(objs.reshape(-1, 1).astype(jnp.int32),
                                     TsPad, ToPad)
    h0 = _pred_h0(p.reshape(-1, 1).astype(jnp.int32), PpPad, ba0)

    SQ, r = _edge_pass(As_node, Ao_node, s, o, h0, zerosGH,
                       want_r=True, tag="l0")

    for ell in range(1, NUM_LAYERS + 1):
        prev = Wsplit[ell - 1]
        if ell < NUM_LAYERS:
            cur = Wsplit[ell]
            Bs, Bo = _node_update(
                SQ[0][:O_NODES], SQ[1][:O_NODES], csb, cob, prev["W2a"], prev["W2c"],
                prev["b2a"], prev["b2c"],
                gconv[ell - 1]["net2"][0][0], gconv[ell - 1]["net2"][0][1],
                gconv[ell - 1]["net2"][1][0], gconv[ell - 1]["net2"][1][1],
                cur["W1s"], jnp.zeros((GH,), jnp.float32),
                cur["W1o"], jnp.zeros((GH,), jnp.float32),
                final=False, tag=f"l{ell - 1}")
            M = prev["W2b"] @ cur["W1p"]
            cvec = cur["b1"] + prev["b2b"] @ cur["W1p"]
            h = _h_matmul(r, M, cvec)
            want_r = ell < NUM_LAYERS - 1
            res = _edge_pass(Bs, Bo, s, o, h, zerosGH,
                             want_r=want_r, tag=f"l{ell}")
            if want_r:
                SQ, r = res
            else:
                (SQ,) = res
        else:
            (U1, e1), (U2, e2) = params["box"]
            U2p = jnp.zeros((GH, GH), jnp.float32).at[:, :4].set(U2)
            e2p = jnp.zeros((GH,), jnp.float32).at[:4].set(e2)
            (boxes_pad,) = _node_update(
                SQ[0][:O_NODES], SQ[1][:O_NODES], csb, cob, prev["W2a"], prev["W2c"],
                prev["b2a"], prev["b2c"],
                gconv[ell - 1]["net2"][0][0], gconv[ell - 1]["net2"][0][1],
                gconv[ell - 1]["net2"][1][0], gconv[ell - 1]["net2"][1][1],
                U1, e1, U2p, e2p, final=True, tag="box")
            return boxes_pad[:, :4]
